# baseline (speedup 1.0000x reference)
import functools

import jax
import jax.numpy as jnp
from jax import lax
from jax.experimental import pallas as pl
from jax.experimental.pallas import tpu as pltpu
from jax.experimental.pallas import tpu_sc as plsc

NC, NS = 2, 16
NW = NC * NS
HIST = 200
BATCH = 4096
EMB = 64
OUT = 46
BPW = BATCH // NW
LANES = EMB // 16


def _emb_bag_sc(seq, emb_table):
    mesh = plsc.VectorSubcoreMesh(core_axis_name="c", subcore_axis_name="s")

    @functools.partial(
        pl.kernel,
        out_type=jax.ShapeDtypeStruct((BATCH, EMB), jnp.float32),
        mesh=mesh,
        compiler_params=pltpu.CompilerParams(use_tc_tiling_on_sc=False),
        scratch_types=[
            pltpu.VMEM((HIST, BPW), jnp.int32),
            pltpu.VMEM((BPW, EMB), jnp.float32),
            pltpu.VMEM((BPW, EMB), jnp.float32),
            pltpu.VMEM((BPW, EMB), jnp.float32),
            pltpu.SemaphoreType.DMA,
            pltpu.SemaphoreType.DMA,
        ],
    )
    def bag(seq_hbm, table_hbm, out_hbm, idx_v, rows0, rows1, acc, sem0, sem1):
        wid = lax.axis_index("s") * NC + lax.axis_index("c")
        base = wid * BPW
        pltpu.sync_copy(seq_hbm.at[:, pl.ds(base, BPW)], idx_v)

        pltpu.async_copy(table_hbm.at[idx_v.at[0]], rows0, sem0)
        zeros = jnp.zeros((16,), jnp.float32)

        @pl.loop(0, BPW)
        def _(j):
            for k in range(LANES):
                acc[j, pl.ds(k * 16, 16)] = zeros

        def accumulate(rows):
            @pl.loop(0, BPW, unroll=4)
            def _(j):
                for k in range(LANES):
                    sl = pl.ds(k * 16, 16)
                    plsc.addupdate(acc.at[j, sl], rows[j, sl])

        @pl.loop(0, HIST - 2, step=2)
        def _(h):
            pltpu.make_async_copy(table_hbm.at[idx_v.at[h]], rows0, sem0).wait()
            d1 = pltpu.async_copy(table_hbm.at[idx_v.at[h + 1]], rows1, sem1)
            accumulate(rows0)
            d1.wait()
            pltpu.async_copy(table_hbm.at[idx_v.at[h + 2]], rows0, sem0)
            accumulate(rows1)

        pltpu.make_async_copy(
            table_hbm.at[idx_v.at[HIST - 2]], rows0, sem0).wait()
        dl = pltpu.async_copy(table_hbm.at[idx_v.at[HIST - 1]], rows1, sem1)
        accumulate(rows0)
        dl.wait()
        accumulate(rows1)

        pltpu.sync_copy(acc, out_hbm.at[pl.ds(base, BPW)])

    return bag(seq, emb_table)


VOCAB = 1000000
TR_BLK = 4096


def _relayout_tc(table_t):
    def tr(x_ref, o_ref):
        o_ref[...] = x_ref[...].T

    return pl.pallas_call(
        tr,
        out_shape=jax.ShapeDtypeStruct((VOCAB, EMB), jnp.float32),
        grid=(pl.cdiv(VOCAB, TR_BLK),),
        in_specs=[pl.BlockSpec((EMB, TR_BLK), lambda i: (0, i))],
        out_specs=pl.BlockSpec((TR_BLK, EMB), lambda i: (i, 0)),
    )(table_t)


def _linear_tc(emb, W, b2):
    def mm(x_ref, w_ref, b_ref, o_ref):
        o_ref[...] = lax.dot_general(
            x_ref[...], w_ref[...],
            (((1,), (1,)), ((), ())),
            preferred_element_type=jnp.float32,
        ) + b_ref[...]

    return pl.pallas_call(
        mm,
        out_shape=jax.ShapeDtypeStruct((BATCH, OUT), jnp.float32),
    )(emb, W, b2)


def kernel(seq, emb_table, W, b):
    table_rm = _relayout_tc(emb_table.T)
    emb = _emb_bag_sc(seq.astype(jnp.int32), table_rm)
    return _linear_tc(emb, W, b.reshape(1, OUT))


# speedup vs baseline: 1.1469x; 1.1469x over previous
import functools

import jax
import jax.numpy as jnp
from jax import lax
from jax.experimental import pallas as pl
from jax.experimental.pallas import tpu as pltpu
from jax.experimental.pallas import tpu_sc as plsc

NC, NS = 2, 16
NW = NC * NS
HIST = 200
BATCH = 4096
EMB = 64
OUT = 46
BPW = BATCH // NW
LANES = EMB // 16
CHUNK = 50
PAIRS = CHUNK // 2


def _emb_bag_sc(seq, table):
    mesh = plsc.VectorSubcoreMesh(core_axis_name="c", subcore_axis_name="s")

    @functools.partial(
        pl.kernel,
        out_type=jax.ShapeDtypeStruct((BATCH, EMB), jnp.float32),
        mesh=mesh,
        compiler_params=pltpu.CompilerParams(use_tc_tiling_on_sc=False),
        scratch_types=[
            pltpu.VMEM((CHUNK, BPW), jnp.int32),
            pltpu.VMEM((BPW, EMB), jnp.float32),
            pltpu.VMEM((BPW, EMB), jnp.float32),
            pltpu.VMEM((BPW, EMB), jnp.float32),
            pltpu.SemaphoreType.DMA,
            pltpu.SemaphoreType.DMA,
        ],
    )
    def bag(seq_hbm, table_hbm, out_hbm, idx_v, rows0, rows1, acc, sem0, sem1):
        wid = lax.axis_index("s") * NC + lax.axis_index("c")
        base = wid * BPW

        zeros = jnp.zeros((16,), jnp.float32)

        @pl.loop(0, BPW)
        def _(j):
            for k in range(LANES):
                acc[j, pl.ds(k * 16, 16)] = zeros

        def accumulate(rows):
            @pl.loop(0, BPW, unroll=4)
            def _(j):
                for k in range(LANES):
                    sl = pl.ds(k * 16, 16)
                    plsc.addupdate(acc.at[j, sl], rows[j, sl])

        @pl.loop(0, HIST // CHUNK)
        def _(c):
            pltpu.sync_copy(
                seq_hbm.at[pl.ds(c * CHUNK, CHUNK), pl.ds(base, BPW)], idx_v)
            pltpu.async_copy(table_hbm.at[idx_v.at[0]], rows0, sem0)

            @pl.loop(0, PAIRS - 1)
            def _(g):
                h = 2 * g
                pltpu.make_async_copy(
                    table_hbm.at[idx_v.at[h]], rows0, sem0).wait()
                d1 = pltpu.async_copy(
                    table_hbm.at[idx_v.at[h + 1]], rows1, sem1)
                accumulate(rows0)
                d1.wait()
                pltpu.async_copy(table_hbm.at[idx_v.at[h + 2]], rows0, sem0)
                accumulate(rows1)

            pltpu.make_async_copy(
                table_hbm.at[idx_v.at[CHUNK - 2]], rows0, sem0).wait()
            dl = pltpu.async_copy(
                table_hbm.at[idx_v.at[CHUNK - 1]], rows1, sem1)
            accumulate(rows0)
            dl.wait()
            accumulate(rows1)

        pltpu.sync_copy(acc, out_hbm.at[pl.ds(base, BPW)])

    return bag(seq, table)


def _linear_tc(emb, W, b2):
    def mm(x_ref, w_ref, b_ref, o_ref):
        o_ref[...] = lax.dot_general(
            x_ref[...], w_ref[...],
            (((1,), (1,)), ((), ())),
            preferred_element_type=jnp.float32,
        ) + b_ref[...]

    return pl.pallas_call(
        mm,
        out_shape=jax.ShapeDtypeStruct((BATCH, OUT), jnp.float32),
    )(emb, W, b2)


def kernel(seq, emb_table, W, b):
    emb = _emb_bag_sc(seq.astype(jnp.int32), emb_table)
    return _linear_tc(emb, W, b.reshape(1, OUT))


# speedup vs baseline: 1.1932x; 1.0403x over previous
import functools

import jax
import jax.numpy as jnp
from jax import lax
from jax.experimental import pallas as pl
from jax.experimental.pallas import tpu as pltpu
from jax.experimental.pallas import tpu_sc as plsc

NC, NS = 2, 16
NW = NC * NS
HIST = 200
BATCH = 4096
EMB = 64
OUT = 46
VOCAB = 1000000
BPW = BATCH // NW
LANES = EMB // 16
CHUNK = 50
PAIRS = CHUNK // 2

TR_BLK = 2048
NTB = -(-VOCAB // (2 * TR_BLK))
PV = NTB * 2 * TR_BLK


def _project_pack_tc(table_t, W):
    def pp(xl_ref, xh_ref, w_ref, o_ref):
        lo = lax.dot_general(
            xl_ref[...], w_ref[...],
            (((0,), (1,)), ((), ())),
            preferred_element_type=jnp.float32,
            precision=lax.Precision.HIGHEST,
        )
        hi = lax.dot_general(
            xh_ref[...], w_ref[...],
            (((0,), (1,)), ((), ())),
            preferred_element_type=jnp.float32,
            precision=lax.Precision.HIGHEST,
        )
        o_ref[...] = jnp.zeros((TR_BLK, 2 * EMB), jnp.float32)
        o_ref[:, 0:OUT] = lo
        o_ref[:, EMB:EMB + OUT] = hi

    return pl.pallas_call(
        pp,
        out_shape=jax.ShapeDtypeStruct((PV // 2, 2 * EMB), jnp.float32),
        grid=(NTB,),
        in_specs=[
            pl.BlockSpec((EMB, TR_BLK), lambda i: (0, 2 * i)),
            pl.BlockSpec(
                (EMB, TR_BLK),
                lambda i: (0, jnp.minimum(2 * i + 1, VOCAB // TR_BLK))),
            pl.BlockSpec((OUT, EMB), lambda i: (0, 0)),
        ],
        out_specs=pl.BlockSpec((TR_BLK, 2 * EMB), lambda i: (i, 0)),
    )(table_t, table_t, W)


def _emb_bag_sc(seq, table):
    mesh = plsc.VectorSubcoreMesh(core_axis_name="c", subcore_axis_name="s")

    @functools.partial(
        pl.kernel,
        out_type=jax.ShapeDtypeStruct((BATCH, EMB), jnp.float32),
        mesh=mesh,
        compiler_params=pltpu.CompilerParams(use_tc_tiling_on_sc=False),
        scratch_types=[
            pltpu.VMEM((CHUNK, BPW), jnp.int32),
            pltpu.VMEM((BPW, EMB), jnp.float32),
            pltpu.VMEM((BPW, EMB), jnp.float32),
            pltpu.VMEM((BPW, EMB), jnp.float32),
            pltpu.SemaphoreType.DMA,
            pltpu.SemaphoreType.DMA,
        ],
    )
    def bag(seq_hbm, table_hbm, out_hbm, idx_v, rows0, rows1, acc, sem0, sem1):
        wid = lax.axis_index("s") * NC + lax.axis_index("c")
        base = wid * BPW

        zeros = jnp.zeros((16,), jnp.float32)

        @pl.loop(0, BPW)
        def _(j):
            for k in range(LANES):
                acc[j, pl.ds(k * 16, 16)] = zeros

        def accumulate(rows):
            @pl.loop(0, BPW, unroll=4)
            def _(j):
                for k in range(LANES):
                    sl = pl.ds(k * 16, 16)
                    plsc.addupdate(acc.at[j, sl], rows[j, sl])

        @pl.loop(0, HIST // CHUNK)
        def _(c):
            pltpu.sync_copy(
                seq_hbm.at[pl.ds(c * CHUNK, CHUNK), pl.ds(base, BPW)], idx_v)

            @pl.loop(0, CHUNK)
            def _(h):
                for k in range(BPW // 16):
                    sl = pl.ds(k * 16, 16)
                    v = idx_v[h, sl]
                    u = jnp.bitwise_and(v, 2 * TR_BLK - 1)
                    idx_v[h, sl] = v + jnp.where(
                        u < TR_BLK, u, u - (2 * TR_BLK - 1))

            pltpu.async_copy(table_hbm.at[idx_v.at[0]], rows0, sem0)

            @pl.loop(0, PAIRS - 1)
            def _(g):
                h = 2 * g
                pltpu.make_async_copy(
                    table_hbm.at[idx_v.at[h]], rows0, sem0).wait()
                d1 = pltpu.async_copy(
                    table_hbm.at[idx_v.at[h + 1]], rows1, sem1)
                accumulate(rows0)
                d1.wait()
                pltpu.async_copy(table_hbm.at[idx_v.at[h + 2]], rows0, sem0)
                accumulate(rows1)

            pltpu.make_async_copy(
                table_hbm.at[idx_v.at[CHUNK - 2]], rows0, sem0).wait()
            dl = pltpu.async_copy(
                table_hbm.at[idx_v.at[CHUNK - 1]], rows1, sem1)
            accumulate(rows0)
            dl.wait()
            accumulate(rows1)

        pltpu.sync_copy(acc, out_hbm.at[pl.ds(base, BPW)])

    return bag(seq, table)


def _bias_slice_tc(bag, b2):
    def bs(x_ref, b_ref, o_ref):
        o_ref[...] = x_ref[:, 0:OUT] + b_ref[...]

    return pl.pallas_call(
        bs,
        out_shape=jax.ShapeDtypeStruct((BATCH, OUT), jnp.float32),
    )(bag, b2)


def kernel(seq, emb_table, W, b):
    packed = _project_pack_tc(emb_table.T, W)
    table_p = packed.reshape(PV, EMB)
    bag = _emb_bag_sc(seq.astype(jnp.int32), table_p)
    return _bias_slice_tc(bag, b.reshape(1, OUT))


# speedup vs baseline: 1.9173x; 1.6069x over previous
import functools

import jax
import jax.numpy as jnp
from jax import lax
from jax.experimental import pallas as pl
from jax.experimental.pallas import tpu as pltpu
from jax.experimental.pallas import tpu_sc as plsc

NC, NS = 2, 16
NW = NC * NS
HIST = 200
BATCH = 4096
EMB = 64
OUT = 46
VOCAB = 1000000
BPW = BATCH // NW
LANES = EMB // 16
CHUNK = 50
PAIRS = CHUNK // 2

TR_BLK = 4096
NTB = -(-VOCAB // (2 * TR_BLK))
PV = NTB * 2 * TR_BLK


def _project_pack_tc(table_t, W):
    def pp(xl_ref, xh_ref, w_ref, o_ref):
        lo = lax.dot_general(
            xl_ref[...], w_ref[...],
            (((0,), (1,)), ((), ())),
            preferred_element_type=jnp.float32,
        )
        hi = lax.dot_general(
            xh_ref[...], w_ref[...],
            (((0,), (1,)), ((), ())),
            preferred_element_type=jnp.float32,
        )
        o_ref[...] = jnp.zeros((TR_BLK, 2 * EMB), jnp.float32)
        o_ref[:, 0:OUT] = lo
        o_ref[:, EMB:EMB + OUT] = hi

    return pl.pallas_call(
        pp,
        out_shape=jax.ShapeDtypeStruct((PV // 2, 2 * EMB), jnp.float32),
        grid=(NTB,),
        in_specs=[
            pl.BlockSpec((EMB, TR_BLK), lambda i: (0, 2 * i)),
            pl.BlockSpec(
                (EMB, TR_BLK),
                lambda i: (0, jnp.minimum(2 * i + 1, VOCAB // TR_BLK))),
            pl.BlockSpec((OUT, EMB), lambda i: (0, 0)),
        ],
        out_specs=pl.BlockSpec((TR_BLK, 2 * EMB), lambda i: (i, 0)),
    )(table_t, table_t, W)


def _emb_bag_sc(seq, table):
    mesh = plsc.VectorSubcoreMesh(core_axis_name="c", subcore_axis_name="s")

    @functools.partial(
        pl.kernel,
        out_type=jax.ShapeDtypeStruct((BATCH, EMB), jnp.float32),
        mesh=mesh,
        compiler_params=pltpu.CompilerParams(use_tc_tiling_on_sc=False),
        scratch_types=[
            pltpu.VMEM((CHUNK, BPW), jnp.int32),
            pltpu.VMEM((BPW, EMB), jnp.float32),
            pltpu.VMEM((BPW, EMB), jnp.float32),
            pltpu.VMEM((BPW, EMB), jnp.float32),
            pltpu.SemaphoreType.DMA,
            pltpu.SemaphoreType.DMA,
        ],
    )
    def bag(seq_hbm, table_hbm, out_hbm, idx_v, rows0, rows1, acc, sem0, sem1):
        wid = lax.axis_index("s") * NC + lax.axis_index("c")
        base = wid * BPW

        zeros = jnp.zeros((16,), jnp.float32)

        @pl.loop(0, BPW)
        def _(j):
            for k in range(LANES):
                acc[j, pl.ds(k * 16, 16)] = zeros

        def accumulate(rows):
            @pl.loop(0, BPW, unroll=4)
            def _(j):
                for k in range(LANES):
                    sl = pl.ds(k * 16, 16)
                    plsc.addupdate(acc.at[j, sl], rows[j, sl])

        @pl.loop(0, HIST // CHUNK)
        def _(c):
            pltpu.sync_copy(
                seq_hbm.at[pl.ds(c * CHUNK, CHUNK), pl.ds(base, BPW)], idx_v)

            @pl.loop(0, CHUNK)
            def _(h):
                for k in range(BPW // 16):
                    sl = pl.ds(k * 16, 16)
                    v = idx_v[h, sl]
                    u = jnp.bitwise_and(v, 2 * TR_BLK - 1)
                    idx_v[h, sl] = v + jnp.where(
                        u < TR_BLK, u, u - (2 * TR_BLK - 1))

            pltpu.async_copy(table_hbm.at[idx_v.at[0]], rows0, sem0)

            @pl.loop(0, PAIRS - 1)
            def _(g):
                h = 2 * g
                pltpu.make_async_copy(
                    table_hbm.at[idx_v.at[h]], rows0, sem0).wait()
                d1 = pltpu.async_copy(
                    table_hbm.at[idx_v.at[h + 1]], rows1, sem1)
                accumulate(rows0)
                d1.wait()
                pltpu.async_copy(table_hbm.at[idx_v.at[h + 2]], rows0, sem0)
                accumulate(rows1)

            pltpu.make_async_copy(
                table_hbm.at[idx_v.at[CHUNK - 2]], rows0, sem0).wait()
            dl = pltpu.async_copy(
                table_hbm.at[idx_v.at[CHUNK - 1]], rows1, sem1)
            accumulate(rows0)
            dl.wait()
            accumulate(rows1)

        pltpu.sync_copy(acc, out_hbm.at[pl.ds(base, BPW)])

    return bag(seq, table)


def _bias_slice_tc(bag, b2):
    def bs(x_ref, b_ref, o_ref):
        o_ref[...] = x_ref[:, 0:OUT] + b_ref[...]

    return pl.pallas_call(
        bs,
        out_shape=jax.ShapeDtypeStruct((BATCH, OUT), jnp.float32),
    )(bag, b2)


def kernel(seq, emb_table, W, b):
    packed = _project_pack_tc(emb_table.T, W)
    table_p = packed.reshape(PV, EMB)
    bag = _emb_bag_sc(seq.astype(jnp.int32), table_p)
    return _bias_slice_tc(bag, b.reshape(1, OUT))


# speedup vs baseline: 2.0495x; 1.0689x over previous
import functools

import jax
import jax.numpy as jnp
from jax import lax
from jax.experimental import pallas as pl
from jax.experimental.pallas import tpu as pltpu
from jax.experimental.pallas import tpu_sc as plsc

NC, NS = 2, 16
NW = NC * NS
HIST = 200
BATCH = 4096
EMB = 64
OUT = 46
VOCAB = 1000000
BPW = BATCH // NW
LANES = EMB // 16
CHUNK = 50
PAIRS = CHUNK // 2

TR_BLK = 8192
NTB = -(-VOCAB // (2 * TR_BLK))
PV = NTB * 2 * TR_BLK


def _project_pack_tc(table_t, W):
    def pp(xl_ref, xh_ref, w_ref, o_ref):
        lo = lax.dot_general(
            xl_ref[...], w_ref[...],
            (((0,), (1,)), ((), ())),
            preferred_element_type=jnp.float32,
        )
        hi = lax.dot_general(
            xh_ref[...], w_ref[...],
            (((0,), (1,)), ((), ())),
            preferred_element_type=jnp.float32,
        )
        o_ref[...] = jnp.zeros((TR_BLK, 2 * EMB), jnp.float32)
        o_ref[:, 0:OUT] = lo
        o_ref[:, EMB:EMB + OUT] = hi

    return pl.pallas_call(
        pp,
        out_shape=jax.ShapeDtypeStruct((PV // 2, 2 * EMB), jnp.float32),
        grid=(NTB,),
        in_specs=[
            pl.BlockSpec((EMB, TR_BLK), lambda i: (0, 2 * i)),
            pl.BlockSpec(
                (EMB, TR_BLK),
                lambda i: (0, jnp.minimum(2 * i + 1, VOCAB // TR_BLK))),
            pl.BlockSpec((OUT, EMB), lambda i: (0, 0)),
        ],
        out_specs=pl.BlockSpec((TR_BLK, 2 * EMB), lambda i: (i, 0)),
    )(table_t, table_t, W)


def _emb_bag_sc(seq, table):
    mesh = plsc.VectorSubcoreMesh(core_axis_name="c", subcore_axis_name="s")

    @functools.partial(
        pl.kernel,
        out_type=jax.ShapeDtypeStruct((BATCH, EMB), jnp.float32),
        mesh=mesh,
        compiler_params=pltpu.CompilerParams(use_tc_tiling_on_sc=False),
        scratch_types=[
            pltpu.VMEM((CHUNK, BPW), jnp.int32),
            pltpu.VMEM((BPW, EMB), jnp.float32),
            pltpu.VMEM((BPW, EMB), jnp.float32),
            pltpu.VMEM((BPW, EMB), jnp.float32),
            pltpu.SemaphoreType.DMA,
            pltpu.SemaphoreType.DMA,
        ],
    )
    def bag(seq_hbm, table_hbm, out_hbm, idx_v, rows0, rows1, acc, sem0, sem1):
        wid = lax.axis_index("s") * NC + lax.axis_index("c")
        base = wid * BPW

        zeros = jnp.zeros((16,), jnp.float32)

        @pl.loop(0, BPW)
        def _(j):
            for k in range(LANES):
                acc[j, pl.ds(k * 16, 16)] = zeros

        def accumulate(rows):
            @pl.loop(0, BPW, unroll=4)
            def _(j):
                for k in range(LANES):
                    sl = pl.ds(k * 16, 16)
                    plsc.addupdate(acc.at[j, sl], rows[j, sl])

        @pl.loop(0, HIST // CHUNK)
        def _(c):
            pltpu.sync_copy(
                seq_hbm.at[pl.ds(c * CHUNK, CHUNK), pl.ds(base, BPW)], idx_v)

            @pl.loop(0, CHUNK)
            def _(h):
                for k in range(BPW // 16):
                    sl = pl.ds(k * 16, 16)
                    v = idx_v[h, sl]
                    u = jnp.bitwise_and(v, 2 * TR_BLK - 1)
                    idx_v[h, sl] = v + jnp.where(
                        u < TR_BLK, u, u - (2 * TR_BLK - 1))

            pltpu.async_copy(table_hbm.at[idx_v.at[0]], rows0, sem0)

            @pl.loop(0, PAIRS - 1)
            def _(g):
                h = 2 * g
                pltpu.make_async_copy(
                    table_hbm.at[idx_v.at[h]], rows0, sem0).wait()
                d1 = pltpu.async_copy(
                    table_hbm.at[idx_v.at[h + 1]], rows1, sem1)
                accumulate(rows0)
                d1.wait()
                pltpu.async_copy(table_hbm.at[idx_v.at[h + 2]], rows0, sem0)
                accumulate(rows1)

            pltpu.make_async_copy(
                table_hbm.at[idx_v.at[CHUNK - 2]], rows0, sem0).wait()
            dl = pltpu.async_copy(
                table_hbm.at[idx_v.at[CHUNK - 1]], rows1, sem1)
            accumulate(rows0)
            dl.wait()
            accumulate(rows1)

        pltpu.sync_copy(acc, out_hbm.at[pl.ds(base, BPW)])

    return bag(seq, table)


def _bias_slice_tc(bag, b2):
    def bs(x_ref, b_ref, o_ref):
        o_ref[...] = x_ref[:, 0:OUT] + b_ref[...]

    return pl.pallas_call(
        bs,
        out_shape=jax.ShapeDtypeStruct((BATCH, OUT), jnp.float32),
    )(bag, b2)


def kernel(seq, emb_table, W, b):
    packed = _project_pack_tc(emb_table.T, W)
    table_p = packed.reshape(PV, EMB)
    bag = _emb_bag_sc(seq.astype(jnp.int32), table_p)
    return _bias_slice_tc(bag, b.reshape(1, OUT))


# speedup vs baseline: 2.0716x; 1.0108x over previous
import functools

import jax
import jax.numpy as jnp
from jax import lax
from jax.experimental import pallas as pl
from jax.experimental.pallas import tpu as pltpu
from jax.experimental.pallas import tpu_sc as plsc

NC, NS = 2, 16
NW = NC * NS
HIST = 200
BATCH = 4096
EMB = 64
OUT = 46
VOCAB = 1000000
BPW = BATCH // NW
LANES = EMB // 16
CHUNK = 50
PAIRS = CHUNK // 2

TR_BLK = 16384
NTB = -(-VOCAB // (2 * TR_BLK))
PV = NTB * 2 * TR_BLK


def _project_pack_tc(table_t, W):
    def pp(xl_ref, xh_ref, w_ref, o_ref):
        lo = lax.dot_general(
            xl_ref[...], w_ref[...],
            (((0,), (1,)), ((), ())),
            preferred_element_type=jnp.float32,
        )
        hi = lax.dot_general(
            xh_ref[...], w_ref[...],
            (((0,), (1,)), ((), ())),
            preferred_element_type=jnp.float32,
        )
        o_ref[...] = jnp.zeros((TR_BLK, 2 * EMB), jnp.float32)
        o_ref[:, 0:OUT] = lo
        o_ref[:, EMB:EMB + OUT] = hi

    return pl.pallas_call(
        pp,
        out_shape=jax.ShapeDtypeStruct((PV // 2, 2 * EMB), jnp.float32),
        grid=(NTB,),
        in_specs=[
            pl.BlockSpec((EMB, TR_BLK), lambda i: (0, 2 * i)),
            pl.BlockSpec(
                (EMB, TR_BLK),
                lambda i: (0, jnp.minimum(2 * i + 1, VOCAB // TR_BLK))),
            pl.BlockSpec((OUT, EMB), lambda i: (0, 0)),
        ],
        out_specs=pl.BlockSpec((TR_BLK, 2 * EMB), lambda i: (i, 0)),
    )(table_t, table_t, W)


def _emb_bag_sc(seq, table):
    mesh = plsc.VectorSubcoreMesh(core_axis_name="c", subcore_axis_name="s")

    @functools.partial(
        pl.kernel,
        out_type=jax.ShapeDtypeStruct((BATCH, EMB), jnp.float32),
        mesh=mesh,
        compiler_params=pltpu.CompilerParams(use_tc_tiling_on_sc=False),
        scratch_types=[
            pltpu.VMEM((CHUNK, BPW), jnp.int32),
            pltpu.VMEM((BPW, EMB), jnp.float32),
            pltpu.VMEM((BPW, EMB), jnp.float32),
            pltpu.VMEM((BPW, EMB), jnp.float32),
            pltpu.SemaphoreType.DMA,
            pltpu.SemaphoreType.DMA,
        ],
    )
    def bag(seq_hbm, table_hbm, out_hbm, idx_v, rows0, rows1, acc, sem0, sem1):
        wid = lax.axis_index("s") * NC + lax.axis_index("c")
        base = wid * BPW

        zeros = jnp.zeros((16,), jnp.float32)

        @pl.loop(0, BPW)
        def _(j):
            for k in range(LANES):
                acc[j, pl.ds(k * 16, 16)] = zeros

        def accumulate(rows):
            @pl.loop(0, BPW, unroll=4)
            def _(j):
                for k in range(LANES):
                    sl = pl.ds(k * 16, 16)
                    plsc.addupdate(acc.at[j, sl], rows[j, sl])

        @pl.loop(0, HIST // CHUNK)
        def _(c):
            pltpu.sync_copy(
                seq_hbm.at[pl.ds(c * CHUNK, CHUNK), pl.ds(base, BPW)], idx_v)

            @pl.loop(0, CHUNK)
            def _(h):
                for k in range(BPW // 16):
                    sl = pl.ds(k * 16, 16)
                    v = idx_v[h, sl]
                    u = jnp.bitwise_and(v, 2 * TR_BLK - 1)
                    idx_v[h, sl] = v + jnp.where(
                        u < TR_BLK, u, u - (2 * TR_BLK - 1))

            pltpu.async_copy(table_hbm.at[idx_v.at[0]], rows0, sem0)

            @pl.loop(0, PAIRS - 1)
            def _(g):
                h = 2 * g
                pltpu.make_async_copy(
                    table_hbm.at[idx_v.at[h]], rows0, sem0).wait()
                d1 = pltpu.async_copy(
                    table_hbm.at[idx_v.at[h + 1]], rows1, sem1)
                accumulate(rows0)
                d1.wait()
                pltpu.async_copy(table_hbm.at[idx_v.at[h + 2]], rows0, sem0)
                accumulate(rows1)

            pltpu.make_async_copy(
                table_hbm.at[idx_v.at[CHUNK - 2]], rows0, sem0).wait()
            dl = pltpu.async_copy(
                table_hbm.at[idx_v.at[CHUNK - 1]], rows1, sem1)
            accumulate(rows0)
            dl.wait()
            accumulate(rows1)

        pltpu.sync_copy(acc, out_hbm.at[pl.ds(base, BPW)])

    return bag(seq, table)


def _bias_slice_tc(bag, b2):
    def bs(x_ref, b_ref, o_ref):
        o_ref[...] = x_ref[:, 0:OUT] + b_ref[...]

    return pl.pallas_call(
        bs,
        out_shape=jax.ShapeDtypeStruct((BATCH, OUT), jnp.float32),
    )(bag, b2)


def kernel(seq, emb_table, W, b):
    packed = _project_pack_tc(emb_table.T, W)
    table_p = packed.reshape(PV, EMB)
    bag = _emb_bag_sc(seq.astype(jnp.int32), table_p)
    return _bias_slice_tc(bag, b.reshape(1, OUT))
